# baseline (speedup 1.0000x reference)
import math

import jax
import jax.numpy as jnp
import numpy as np
from jax.experimental import pallas as pl

try:
    from jax.experimental.shard_map import shard_map as _shard_map
except ImportError:
    _shard_map = jax.shard_map

_B, _N, _D = 8, 2048, 3
_K = 16
_NSTRIP = 8
_S = _N // _NSTRIP


def _hi_lo(x):
    hi = x.astype(jnp.bfloat16)
    lo = (x - hi.astype(jnp.float32)).astype(jnp.bfloat16)
    return hi, lo


def _chamfer_body(a_ref, g_ref, out_ref, *, nb):
    s = None
    for b in range(nb):
        at = a_ref[b]
        gt = g_ref[b]

        a2h, a2l = _hi_lo(at * (-2.0))
        bh, bl = _hi_lo(gt)
        nah, nal = _hi_lo(jnp.sum(at * at, axis=0, keepdims=True))
        nbh, nbl = _hi_lo(jnp.sum(gt * gt, axis=0, keepdims=True))
        one = jnp.ones((1, _N), jnp.bfloat16)

        aa = jnp.concatenate([a2h, a2l, a2h, a2l, one, one, nah, nal],
                             axis=0)
        bb = jnp.concatenate([bh, bh, bl, bl, nbh, nbl, one, one], axis=0)

        srow = None
        cmin = None
        for r in range(_NSTRIP):
            a_strip = aa[:, r * _S:(r + 1) * _S]
            d = jax.lax.dot_general(
                a_strip, bb,
                dimension_numbers=(((0,), (0,)), ((), ())),
                preferred_element_type=jnp.float32,
            )
            rmin = jnp.min(d, axis=1, keepdims=True)
            cpart = jnp.min(d, axis=0, keepdims=True)
            sr = jnp.sum(rmin, axis=(0, 1), keepdims=True)
            srow = sr if srow is None else srow + sr
            cmin = cpart if cmin is None else jnp.minimum(cmin, cpart)

        sb = srow + jnp.sum(cmin, axis=(0, 1), keepdims=True)
        s = sb if s is None else s + sb

    out_ref[:, :] = s


def _partial_sum(at, gt, nb):
    import functools
    out = pl.pallas_call(
        functools.partial(_chamfer_body, nb=nb),
        in_specs=[
            pl.BlockSpec((nb, _D, _N), lambda: (0, 0, 0)),
            pl.BlockSpec((nb, _D, _N), lambda: (0, 0, 0)),
        ],
        out_specs=pl.BlockSpec((1, 1), lambda: (0, 0)),
        out_shape=jax.ShapeDtypeStruct((1, 1), jnp.float32),
    )(at, gt)
    return out


def kernel(predict_pc, gt_pc):
    at = jnp.transpose(predict_pc, (0, 2, 1))
    gt = jnp.transpose(gt_pc, (0, 2, 1))

    devs = jax.devices()
    nshard = math.gcd(_B, len(devs))
    if nshard > 1:
        mesh = jax.sharding.Mesh(np.array(devs[:nshard]), ("x",))
        spec = jax.sharding.PartitionSpec

        def _shard_fn(a_s, g_s):
            part = _partial_sum(a_s, g_s, _B // nshard)
            return jax.lax.psum(part, "x")

        kw = dict(mesh=mesh, in_specs=(spec("x"), spec("x")),
                  out_specs=spec())
        try:
            fn = _shard_map(_shard_fn, check_vma=False, **kw)
        except TypeError:
            fn = _shard_map(_shard_fn, check_rep=False, **kw)
        out = fn(at, gt)
    else:
        out = _partial_sum(at, gt, _B)
    return out[0, 0] / (2.0 * _B * _N)


# speedup vs baseline: 19.5119x; 19.5119x over previous
import functools

import jax
import jax.numpy as jnp
from jax.experimental import pallas as pl

_B, _N, _D = 8, 2048, 3
_K = 16
_NSTRIP = 8
_S = _N // _NSTRIP


def _hi_lo(x):
    hi = x.astype(jnp.bfloat16)
    lo = (x - hi.astype(jnp.float32)).astype(jnp.bfloat16)
    return hi, lo


def _chamfer_body(a_ref, g_ref, out_ref, *, nb):
    s = None
    for b in range(nb):
        at = a_ref[b]
        gt = g_ref[b]

        a2h, a2l = _hi_lo(at * (-2.0))
        bh, bl = _hi_lo(gt)
        nah, nal = _hi_lo(jnp.sum(at * at, axis=0, keepdims=True))
        nbh, nbl = _hi_lo(jnp.sum(gt * gt, axis=0, keepdims=True))
        one = jnp.ones((1, _N), jnp.bfloat16)

        aa = jnp.concatenate([a2h, a2l, a2h, a2l, one, one, nah, nal],
                             axis=0)
        bb = jnp.concatenate([bh, bh, bl, bl, nbh, nbl, one, one], axis=0)

        srow = None
        cmin = None
        for r in range(_NSTRIP):
            a_strip = aa[:, r * _S:(r + 1) * _S]
            d = jax.lax.dot_general(
                a_strip, bb,
                dimension_numbers=(((0,), (0,)), ((), ())),
                preferred_element_type=jnp.float32,
            )
            rmin = jnp.min(d, axis=1, keepdims=True)
            cpart = jnp.min(d, axis=0, keepdims=True)
            sr = jnp.sum(rmin, axis=(0, 1), keepdims=True)
            srow = sr if srow is None else srow + sr
            cmin = cpart if cmin is None else jnp.minimum(cmin, cpart)

        sb = srow + jnp.sum(cmin, axis=(0, 1), keepdims=True)
        s = sb if s is None else s + sb

    out_ref[:, :] = s


def _partial_sum(at, gt, nb):
    out = pl.pallas_call(
        functools.partial(_chamfer_body, nb=nb),
        in_specs=[
            pl.BlockSpec((nb, _D, _N), lambda: (0, 0, 0)),
            pl.BlockSpec((nb, _D, _N), lambda: (0, 0, 0)),
        ],
        out_specs=pl.BlockSpec((1, 1), lambda: (0, 0)),
        out_shape=jax.ShapeDtypeStruct((1, 1), jnp.float32),
    )(at, gt)
    return out


def kernel(predict_pc, gt_pc):
    at = jnp.transpose(predict_pc, (0, 2, 1))
    gt = jnp.transpose(gt_pc, (0, 2, 1))

    out = _partial_sum(at, gt, _B)
    return out[0, 0] / (2.0 * _B * _N)
